# baseline (speedup 1.0000x reference)
import functools

import jax
import jax.numpy as jnp
from jax import lax
from jax.experimental import pallas as pl
from jax.experimental.pallas import tpu as pltpu
from jax.experimental.pallas import tpu_sc as plsc

N_NODES = 10000
N_PAD = 10240
N_EDGES = 320000
E_PAD = 327680
NC = 2
NS = 16
NW = NC * NS
E_PER_W = E_PAD // NW
CHUNK = 128
N_CHUNKS = E_PER_W // CHUNK
STRIPE = N_PAD // NS
D1 = 128
D2 = 64
NUM_CLASSES_OUT = 40

_mesh = functools.partial(
    plsc.VectorSubcoreMesh, core_axis_name="c", subcore_axis_name="s")
_SC_PARAMS = pltpu.CompilerParams(needs_layout_passes=False)
_SC_AGG_PARAMS = pltpu.CompilerParams(
    needs_layout_passes=False, use_tc_tiling_on_sc=False)


@functools.partial(
    pl.kernel,
    out_type=jax.ShapeDtypeStruct((NW, N_PAD), jnp.float32),
    mesh=_mesh(),
    compiler_params=_SC_PARAMS,
    scratch_types=[
        pltpu.VMEM((E_PER_W // 128, 128), jnp.int32),
        pltpu.VMEM((N_PAD,), jnp.float32),
    ],
)
def _degree_kernel(dst_hbm, out_hbm, idx_v, deg_v):
    rows = E_PER_W // 128
    c = lax.axis_index("c")
    s = lax.axis_index("s")
    wid = s * NC + c
    pltpu.sync_copy(dst_hbm.at[pl.ds(wid * rows, rows)], idx_v)

    zeros16 = jnp.zeros((16,), jnp.float32)

    def zero_body(i, carry):
        deg_v[pl.ds(pl.multiple_of(i * 16, 16), 16)] = zeros16
        return carry

    lax.fori_loop(0, N_PAD // 16, zero_body, 0)

    ones16 = jnp.ones((16,), jnp.float32)

    def add_body(r, carry):
        for j in range(128 // 16):
            idx16 = idx_v[r, pl.ds(j * 16, 16)]
            plsc.addupdate_scatter(deg_v, [idx16], ones16)
        return carry

    lax.fori_loop(0, rows, add_body, 0)
    pltpu.sync_copy(deg_v, out_hbm.at[wid])


def _make_aggregate_local(chunk=128, nbuf=4, n_spans=2):
    d = 64
    span = E_PER_W // chunk // n_spans

    @functools.partial(
        pl.kernel,
        out_type=jax.ShapeDtypeStruct((NC, N_PAD, d), jnp.float32),
        mesh=_mesh(),
        compiler_params=_SC_AGG_PARAMS,
        scratch_types=[
            pltpu.VMEM((span, chunk), jnp.int32),
            pltpu.VMEM((span, chunk), jnp.int32),
            [pltpu.VMEM((chunk, d), jnp.float32)] * nbuf,
            pltpu.VMEM_SHARED((N_PAD, d), jnp.float32),
            pltpu.VMEM_SHARED((N_PAD, d), jnp.float32),
            [pltpu.SemaphoreType.DMA] * nbuf,
        ],
    )
    def agg(h_hbm, src_hbm, dst_hbm, zero_hbm, out_hbm,
            sidx_v, didx_v, rows_bufs, tab_sh, acc_sh, sems):
        c = lax.axis_index("c")
        s = lax.axis_index("s")
        wid = s * NC + c

        pltpu.sync_copy(h_hbm.at[pl.ds(s * STRIPE, STRIPE)],
                        tab_sh.at[pl.ds(s * STRIPE, STRIPE)])
        pltpu.sync_copy(zero_hbm, acc_sh.at[pl.ds(s * STRIPE, STRIPE)])
        plsc.subcore_barrier()

        def start_gather(k, buf, sem):
            pltpu.async_copy(tab_sh.at[sidx_v.at[k]], buf, sem)

        def wait_gather(buf, sem):
            pltpu.make_async_copy(tab_sh.at[pl.ds(0, chunk)], buf, sem).wait()

        def run_span(row0):
            pltpu.sync_copy(src_hbm.at[pl.ds(row0, span)], sidx_v)
            pltpu.sync_copy(dst_hbm.at[pl.ds(row0, span)], didx_v)
            for b in range(nbuf):
                start_gather(b, rows_bufs[b], sems[b])

            def ring_body(j, carry):
                k0 = j * nbuf
                for b in range(nbuf):
                    k = k0 + b
                    wait_gather(rows_bufs[b], sems[b])
                    pltpu.sync_copy(
                        rows_bufs[b], acc_sh.at[didx_v.at[k]], add=True)

                    @pl.when(k + nbuf < span)
                    def _():
                        start_gather(k + nbuf, rows_bufs[b], sems[b])

                return carry

            lax.fori_loop(0, span // nbuf, ring_body, 0)

        for i in range(n_spans):
            run_span(wid * (E_PER_W // chunk) + i * span)
        plsc.subcore_barrier()

        pltpu.sync_copy(acc_sh.at[pl.ds(s * STRIPE, STRIPE)],
                        out_hbm.at[c, pl.ds(s * STRIPE, STRIPE)])

    return agg


_aggregate_local = _make_aggregate_local()


def _mm_scale_body(parts_ref, x_ref, w_ref, oa_ref, ob_ref, dis_ref):
    deg = jnp.sum(parts_ref[...], axis=0) + 1.0
    dis = 1.0 / jnp.sqrt(deg)
    dis_ref[...] = dis
    h = jnp.dot(x_ref[...], w_ref[...],
                preferred_element_type=jnp.float32,
                precision=lax.Precision.HIGHEST)
    hh = h * dis[:, None]
    oa_ref[...] = hh[:, :D2]
    ob_ref[...] = hh[:, D2:]


def _layer2a_body(pa_ref, ha_ref, dis_ref, b_ref, w_ref, o_ref):
    dis = dis_ref[...][:, None]
    out1a = jnp.maximum(
        (pa_ref[0] + pa_ref[1] + ha_ref[...]) * dis + b_ref[...][None, :],
        0.0)
    o_ref[...] = jnp.dot(out1a, w_ref[...],
                         preferred_element_type=jnp.float32,
                         precision=lax.Precision.HIGHEST)


def _layer2b_body(ga_ref, pb_ref, hb_ref, dis_ref, b_ref, w_ref, o_ref):
    dis = dis_ref[...][:, None]
    out1b = jnp.maximum(
        (pb_ref[0] + pb_ref[1] + hb_ref[...]) * dis + b_ref[...][None, :],
        0.0)
    h2 = ga_ref[...] + jnp.dot(out1b, w_ref[...],
                               preferred_element_type=jnp.float32,
                               precision=lax.Precision.HIGHEST)
    o_ref[...] = h2 * dis


def _final_body(q_ref, hh_ref, dis_ref, b_ref, o_ref):
    acc = q_ref[0] + q_ref[1] + hh_ref[...]
    res = acc * dis_ref[...][:, None]
    o_ref[...] = res[:, :NUM_CLASSES_OUT] + b_ref[...][None, :]


_BLK = 512
_GRID = N_PAD // _BLK
_FBLK = 400


def _row_specs(d):
    return [
        pl.BlockSpec((NC, _BLK, d), lambda i: (0, i, 0)),
        pl.BlockSpec((_BLK, d), lambda i: (i, 0)),
        pl.BlockSpec((_BLK,), lambda i: (i,)),
    ]


def kernel(x, edge_index, W1, b1, W2, b2):
    ei = edge_index.astype(jnp.int32)
    e3 = jnp.pad(ei, ((0, 0), (0, E_PAD - N_EDGES)),
                 constant_values=N_PAD - 1).reshape(2, -1, 128)
    src2, dst2 = e3[0], e3[1]
    xp = jnp.pad(x, ((0, N_PAD - N_NODES), (0, 0)))
    W2p = jnp.pad(W2, ((0, 0), (0, D2 - W2.shape[1])))
    zero2 = jnp.zeros((STRIPE, D2), jnp.float32)

    deg_parts = _degree_kernel(dst2)

    hh1a, hh1b, dis = pl.pallas_call(
        _mm_scale_body,
        grid=(_GRID,),
        in_specs=[
            pl.BlockSpec((NW, _BLK), lambda i: (0, i)),
            pl.BlockSpec((_BLK, D1), lambda i: (i, 0)),
            pl.BlockSpec((D1, D1), lambda i: (0, 0)),
        ],
        out_specs=[
            pl.BlockSpec((_BLK, D2), lambda i: (i, 0)),
            pl.BlockSpec((_BLK, D2), lambda i: (i, 0)),
            pl.BlockSpec((_BLK,), lambda i: (i,)),
        ],
        out_shape=[
            jax.ShapeDtypeStruct((N_PAD, D2), jnp.float32),
            jax.ShapeDtypeStruct((N_PAD, D2), jnp.float32),
            jax.ShapeDtypeStruct((N_PAD,), jnp.float32),
        ],
    )(deg_parts, xp, W1)

    p1a = _aggregate_local(hh1a, src2, dst2, zero2)
    p1b = _aggregate_local(hh1b, src2, dst2, zero2)

    half_specs = [
        pl.BlockSpec((NC, _BLK, D2), lambda i: (0, i, 0)),
        pl.BlockSpec((_BLK, D2), lambda i: (i, 0)),
        pl.BlockSpec((_BLK,), lambda i: (i,)),
        pl.BlockSpec((D2,), lambda i: (0,)),
        pl.BlockSpec((D2, D2), lambda i: (0, 0)),
    ]
    ga = pl.pallas_call(
        _layer2a_body,
        grid=(_GRID,),
        in_specs=half_specs,
        out_specs=pl.BlockSpec((_BLK, D2), lambda i: (i, 0)),
        out_shape=jax.ShapeDtypeStruct((N_PAD, D2), jnp.float32),
    )(p1a, hh1a, dis, b1[:D2], W2p[:D2])

    hh2 = pl.pallas_call(
        _layer2b_body,
        grid=(_GRID,),
        in_specs=[pl.BlockSpec((_BLK, D2), lambda i: (i, 0))] + half_specs,
        out_specs=pl.BlockSpec((_BLK, D2), lambda i: (i, 0)),
        out_shape=jax.ShapeDtypeStruct((N_PAD, D2), jnp.float32),
    )(ga, p1b, hh1b, dis, b1[D2:], W2p[D2:])

    p2 = _aggregate_local(hh2, src2, dst2, zero2)

    out = pl.pallas_call(
        _final_body,
        grid=(_GRID,),
        in_specs=[
            pl.BlockSpec((NC, _BLK, D2), lambda i: (0, i, 0)),
            pl.BlockSpec((_BLK, D2), lambda i: (i, 0)),
            pl.BlockSpec((_BLK,), lambda i: (i,)),
            pl.BlockSpec((NUM_CLASSES_OUT,), lambda i: (0,)),
        ],
        out_specs=pl.BlockSpec((_BLK, NUM_CLASSES_OUT), lambda i: (i, 0)),
        out_shape=jax.ShapeDtypeStruct((N_PAD, NUM_CLASSES_OUT),
                                       jnp.float32),
    )(p2, hh2, dis, b2)

    return out[:N_NODES]


# speedup vs baseline: 1.0443x; 1.0443x over previous
import functools

import jax
import jax.numpy as jnp
from jax import lax
from jax.experimental import pallas as pl
from jax.experimental.pallas import tpu as pltpu
from jax.experimental.pallas import tpu_sc as plsc

N_NODES = 10000
N_PAD = 10240
N_EDGES = 320000
E_PAD = 327680
NC = 2
NS = 16
NW = NC * NS
E_PER_W = E_PAD // NW
CHUNK = 128
N_CHUNKS = E_PER_W // CHUNK
STRIPE = N_PAD // NS
D1 = 128
D2 = 64
NUM_CLASSES_OUT = 40

_mesh = functools.partial(
    plsc.VectorSubcoreMesh, core_axis_name="c", subcore_axis_name="s")
_SC_PARAMS = pltpu.CompilerParams(needs_layout_passes=False)
_SC_AGG_PARAMS = pltpu.CompilerParams(
    needs_layout_passes=False, use_tc_tiling_on_sc=False)


@functools.partial(
    pl.kernel,
    out_type=jax.ShapeDtypeStruct((NW, N_PAD), jnp.float32),
    mesh=_mesh(),
    compiler_params=_SC_PARAMS,
    scratch_types=[
        pltpu.VMEM((E_PER_W // 128, 128), jnp.int32),
        pltpu.VMEM((N_PAD,), jnp.float32),
    ],
)
def _degree_kernel(dst_hbm, out_hbm, idx_v, deg_v):
    rows = E_PER_W // 128
    c = lax.axis_index("c")
    s = lax.axis_index("s")
    wid = s * NC + c
    pltpu.sync_copy(dst_hbm.at[pl.ds(wid * rows, rows)], idx_v)

    zeros16 = jnp.zeros((16,), jnp.float32)

    def zero_body(i, carry):
        deg_v[pl.ds(pl.multiple_of(i * 16, 16), 16)] = zeros16
        return carry

    lax.fori_loop(0, N_PAD // 16, zero_body, 0)

    ones16 = jnp.ones((16,), jnp.float32)

    def add_body(r, carry):
        for j in range(128 // 16):
            idx16 = idx_v[r, pl.ds(j * 16, 16)]
            plsc.addupdate_scatter(deg_v, [idx16], ones16)
        return carry

    lax.fori_loop(0, rows, add_body, 0)
    pltpu.sync_copy(deg_v, out_hbm.at[wid])


def _make_aggregate_local(chunk=128, nbuf=4, n_spans=2):
    d = 64
    span = E_PER_W // chunk // n_spans

    @functools.partial(
        pl.kernel,
        out_type=jax.ShapeDtypeStruct((NC, N_PAD, d), jnp.float32),
        mesh=_mesh(),
        compiler_params=_SC_AGG_PARAMS,
        scratch_types=[
            pltpu.VMEM((span, chunk), jnp.int32),
            pltpu.VMEM((span, chunk), jnp.int32),
            [pltpu.VMEM((chunk, d), jnp.float32)] * nbuf,
            pltpu.VMEM_SHARED((N_PAD, d), jnp.float32),
            pltpu.VMEM_SHARED((N_PAD, d), jnp.float32),
            [pltpu.SemaphoreType.DMA] * nbuf,
        ],
    )
    def agg(h_hbm, src_hbm, dst_hbm, zero_hbm, out_hbm,
            sidx_v, didx_v, rows_bufs, tab_sh, acc_sh, sems):
        c = lax.axis_index("c")
        s = lax.axis_index("s")
        wid = s * NC + c

        pltpu.sync_copy(h_hbm.at[pl.ds(s * STRIPE, STRIPE)],
                        tab_sh.at[pl.ds(s * STRIPE, STRIPE)])
        pltpu.sync_copy(zero_hbm, acc_sh.at[pl.ds(s * STRIPE, STRIPE)])
        plsc.subcore_barrier()

        def start_gather(k, buf, sem):
            pltpu.async_copy(tab_sh.at[sidx_v.at[k]], buf, sem)

        def wait_gather(buf, sem):
            pltpu.make_async_copy(tab_sh.at[pl.ds(0, chunk)], buf, sem).wait()

        def run_span(row0):
            pltpu.sync_copy(src_hbm.at[pl.ds(row0, span)], sidx_v)
            pltpu.sync_copy(dst_hbm.at[pl.ds(row0, span)], didx_v)
            for b in range(nbuf):
                start_gather(b, rows_bufs[b], sems[b])

            def ring_body(j, carry):
                k0 = j * nbuf
                for b in range(nbuf):
                    k = k0 + b
                    wait_gather(rows_bufs[b], sems[b])
                    pltpu.sync_copy(
                        rows_bufs[b], acc_sh.at[didx_v.at[k]], add=True)

                    @pl.when(k + nbuf < span)
                    def _():
                        start_gather(k + nbuf, rows_bufs[b], sems[b])

                return carry

            lax.fori_loop(0, span // nbuf, ring_body, 0)

        for i in range(n_spans):
            run_span(wid * (E_PER_W // chunk) + i * span)
        plsc.subcore_barrier()

        pltpu.sync_copy(acc_sh.at[pl.ds(s * STRIPE, STRIPE)],
                        out_hbm.at[c, pl.ds(s * STRIPE, STRIPE)])

    return agg


_aggregate_local = _make_aggregate_local()


def _mm_scale_body(parts_ref, x_ref, w_ref, oa_ref, ob_ref, dis_ref):
    deg = jnp.sum(parts_ref[...], axis=0) + 1.0
    dis = 1.0 / jnp.sqrt(deg)
    dis_ref[...] = dis
    h = jnp.dot(x_ref[...], w_ref[...],
                preferred_element_type=jnp.float32,
                precision=lax.Precision.HIGHEST)
    hh = h * dis[:, None]
    oa_ref[...] = hh[:, :D2]
    ob_ref[...] = hh[:, D2:]


def _layer2_body(pa_ref, pb_ref, ha_ref, hb_ref, dis_ref, b_ref, w_ref,
                 o_ref):
    dis = dis_ref[...][:, None]
    acc = jnp.concatenate(
        [pa_ref[0] + pa_ref[1] + ha_ref[...],
         pb_ref[0] + pb_ref[1] + hb_ref[...]], axis=1)
    out1 = jnp.maximum(acc * dis + b_ref[...][None, :], 0.0)
    h2 = jnp.dot(out1, w_ref[...],
                 preferred_element_type=jnp.float32,
                 precision=lax.Precision.HIGHEST)
    o_ref[...] = h2 * dis


def _final_body(q_ref, hh_ref, dis_ref, b_ref, o_ref):
    acc = q_ref[0] + q_ref[1] + hh_ref[...]
    res = acc * dis_ref[...][:, None]
    o_ref[...] = res[:, :NUM_CLASSES_OUT] + b_ref[...][None, :]


_BLK = 512
_GRID = N_PAD // _BLK
_FBLK = 400


def _row_specs(d):
    return [
        pl.BlockSpec((NC, _BLK, d), lambda i: (0, i, 0)),
        pl.BlockSpec((_BLK, d), lambda i: (i, 0)),
        pl.BlockSpec((_BLK,), lambda i: (i,)),
    ]


def kernel(x, edge_index, W1, b1, W2, b2):
    ei = edge_index.astype(jnp.int32)
    e3 = jnp.pad(ei, ((0, 0), (0, E_PAD - N_EDGES)),
                 constant_values=N_PAD - 1).reshape(2, -1, 128)
    src2, dst2 = e3[0], e3[1]
    xp = jnp.pad(x, ((0, N_PAD - N_NODES), (0, 0)))
    W2p = jnp.pad(W2, ((0, 0), (0, D2 - W2.shape[1])))
    zero2 = jnp.zeros((STRIPE, D2), jnp.float32)

    deg_parts = _degree_kernel(dst2)

    hh1a, hh1b, dis = pl.pallas_call(
        _mm_scale_body,
        grid=(_GRID,),
        in_specs=[
            pl.BlockSpec((NW, _BLK), lambda i: (0, i)),
            pl.BlockSpec((_BLK, D1), lambda i: (i, 0)),
            pl.BlockSpec((D1, D1), lambda i: (0, 0)),
        ],
        out_specs=[
            pl.BlockSpec((_BLK, D2), lambda i: (i, 0)),
            pl.BlockSpec((_BLK, D2), lambda i: (i, 0)),
            pl.BlockSpec((_BLK,), lambda i: (i,)),
        ],
        out_shape=[
            jax.ShapeDtypeStruct((N_PAD, D2), jnp.float32),
            jax.ShapeDtypeStruct((N_PAD, D2), jnp.float32),
            jax.ShapeDtypeStruct((N_PAD,), jnp.float32),
        ],
    )(deg_parts, xp, W1)

    p1a = _aggregate_local(hh1a, src2, dst2, zero2)
    p1b = _aggregate_local(hh1b, src2, dst2, zero2)

    hh2 = pl.pallas_call(
        _layer2_body,
        grid=(_GRID,),
        in_specs=[
            pl.BlockSpec((NC, _BLK, D2), lambda i: (0, i, 0)),
            pl.BlockSpec((NC, _BLK, D2), lambda i: (0, i, 0)),
            pl.BlockSpec((_BLK, D2), lambda i: (i, 0)),
            pl.BlockSpec((_BLK, D2), lambda i: (i, 0)),
            pl.BlockSpec((_BLK,), lambda i: (i,)),
            pl.BlockSpec((D1,), lambda i: (0,)),
            pl.BlockSpec((D1, D2), lambda i: (0, 0)),
        ],
        out_specs=pl.BlockSpec((_BLK, D2), lambda i: (i, 0)),
        out_shape=jax.ShapeDtypeStruct((N_PAD, D2), jnp.float32),
    )(p1a, p1b, hh1a, hh1b, dis, b1, W2p)

    p2 = _aggregate_local(hh2, src2, dst2, zero2)

    out = pl.pallas_call(
        _final_body,
        grid=(_GRID,),
        in_specs=[
            pl.BlockSpec((NC, _BLK, D2), lambda i: (0, i, 0)),
            pl.BlockSpec((_BLK, D2), lambda i: (i, 0)),
            pl.BlockSpec((_BLK,), lambda i: (i,)),
            pl.BlockSpec((NUM_CLASSES_OUT,), lambda i: (0,)),
        ],
        out_specs=pl.BlockSpec((_BLK, NUM_CLASSES_OUT), lambda i: (i, 0)),
        out_shape=jax.ShapeDtypeStruct((N_PAD, NUM_CLASSES_OUT),
                                       jnp.float32),
    )(p2, hh2, dis, b2)

    return out[:N_NODES]
